# Initial kernel scaffold; baseline (speedup 1.0000x reference)
import jax
import jax.numpy as jnp
from jax.experimental import pallas as pl


def kernel(edge_features, cell_0_bounds):
    raise NotImplementedError("write your pallas kernel here")


# speedup vs baseline: 262.9904x; 262.9904x over previous
import functools

import jax
import jax.numpy as jnp
from jax import lax
from jax.experimental import pallas as pl
from jax.experimental.pallas import tpu as pltpu
from jax.experimental.pallas import tpu_sc as plsc

_NW = 32
_CHUNK = 512
_CPAD = 64


def _make_sc_pool(E, D, SPW, BCNT):
    NC = 2
    EP = E // 8
    CP = (_CHUNK + _CPAD) // 8
    OROWS = SPW // 4
    mesh = plsc.VectorSubcoreMesh(core_axis_name="c", subcore_axis_name="s")

    @functools.partial(
        pl.kernel,
        mesh=mesh,
        out_type=jax.ShapeDtypeStruct((_NW * OROWS, 128), jnp.float32),
        scratch_types=[
            pltpu.VMEM((CP, 128), jnp.float32),
            pltpu.VMEM((BCNT,), jnp.int32),
            pltpu.VMEM((OROWS, 128), jnp.float32),
        ],
    )
    def pool(edges_hbm, bounds_hbm, out_hbm, buf, bvm, obuf):
        w = lax.axis_index("s") * NC + lax.axis_index("c")
        seg0 = w * SPW
        start8 = (seg0 // 8) * 8
        off = seg0 - start8
        pltpu.sync_copy(
            bounds_hbm.at[pl.ds(pl.multiple_of(start8, 8), BCNT)], bvm)

        def bload(i):
            return bvm[pl.ds(i, 16)][0]

        r_lo = bload(off)
        r_hi = bload(off + SPW)
        nrows = r_hi - r_lo
        nchunks = jnp.maximum((nrows + _CHUNK - 1) // _CHUNK, 1)

        pos_inf = jnp.full((16,), jnp.inf, jnp.float32)
        neg_inf = jnp.full((16,), -jnp.inf, jnp.float32)
        nbits = max(1, (SPW + 1).bit_length())

        def chunk_body(k, carry):
            s_cur, mn, mx = carry
            base = r_lo + k * _CHUNK
            base_c = jnp.minimum((base // _CPAD) * _CPAD,
                                 E - _CHUNK - _CPAD)
            shift = base - base_c
            pstart = pl.multiple_of(base_c // 8, 8)
            pltpu.sync_copy(edges_hbm.at[pl.ds(pstart, CP)], buf)
            n = jnp.maximum(jnp.minimum(_CHUNK, r_hi - base), 0)
            limit = base + n

            def bs_body(_, lohi):
                lo, hi = lohi
                mid = (lo + hi) // 2
                c = bload(off + mid) <= limit
                return jnp.where(c, mid, lo), jnp.where(c, hi, mid)

            t_max, _ = lax.fori_loop(
                0, nbits, bs_body, (jnp.int32(0), jnp.int32(SPW + 1)))

            def row_body(j, acc):
                a, b = acc
                r = shift + j
                v = buf[r // 8, pl.ds((r % 8) * 16, 16)]
                return jnp.minimum(a, v), jnp.maximum(b, v)

            def seg_body(s, st):
                i, mn, mx = st
                hi_local = bload(off + s + 1) - base
                mn, mx = lax.fori_loop(i, hi_local, row_body, (mn, mx))
                obuf[s // 4, pl.ds((s % 4) * 32, 16)] = mn
                obuf[s // 4, pl.ds((s % 4) * 32 + 16, 16)] = mx
                return hi_local, pos_inf, neg_inf

            i, mn, mx = lax.fori_loop(
                s_cur, t_max, seg_body, (jnp.int32(0), mn, mx))
            mn, mx = lax.fori_loop(i, n, row_body, (mn, mx))
            return t_max, mn, mx

        init = (jnp.int32(0), pos_inf, neg_inf)
        lax.fori_loop(0, nchunks, chunk_body, init)
        pltpu.sync_copy(
            obuf, out_hbm.at[pl.ds(pl.multiple_of(w * OROWS, 8), OROWS)])

    return pool


def kernel(edge_features, cell_0_bounds):
    E, D = edge_features.shape
    S = cell_0_bounds.shape[0] - 1
    assert D == 16 and E % _CPAD == 0
    SPW = ((-(-S // _NW) + 31) // 32) * 32
    S_pad = _NW * SPW
    BCNT = ((SPW + 8) // 8 + 1) * 8 + 16
    pad_len = (S_pad + 48) - (S + 1)
    bounds = jnp.concatenate(
        [cell_0_bounds.astype(jnp.int32),
         jnp.full((pad_len,), E, jnp.int32)])
    edges_packed = edge_features.reshape(E // 8, 8 * D)
    out = _make_sc_pool(E, D, SPW, BCNT)(edges_packed, bounds)
    return out.reshape(S_pad, 2 * D)[:S]


# speedup vs baseline: 338.9650x; 1.2889x over previous
import functools

import jax
import jax.numpy as jnp
from jax import lax
from jax.experimental import pallas as pl
from jax.experimental.pallas import tpu as pltpu
from jax.experimental.pallas import tpu_sc as plsc

_NW = 32
_C = 288
_CB = _C + 8


def _make_sc_pool(E, SPW, BCNT):
    NC = 2
    H = SPW // 2
    HR = H // 4
    OROWS = SPW // 4
    mesh = plsc.VectorSubcoreMesh(core_axis_name="c", subcore_axis_name="s")

    @functools.partial(
        pl.kernel,
        mesh=mesh,
        out_type=jax.ShapeDtypeStruct((_NW * OROWS, 128), jnp.float32),
        scratch_types=[
            pltpu.VMEM((_CB, 16), jnp.float32),
            pltpu.VMEM((_CB, 16), jnp.float32),
            pltpu.VMEM((BCNT,), jnp.int32),
            pltpu.VMEM((HR, 128), jnp.float32),
            pltpu.SemaphoreType.DMA,
            pltpu.SemaphoreType.DMA,
        ],
    )
    def pool(edges_hbm, bounds_hbm, out_hbm, buf0, buf1, bvm, obuf,
             sem0, sem1):
        w = lax.axis_index("s") * NC + lax.axis_index("c")
        seg0 = w * SPW
        start8 = (seg0 // 8) * 8
        off = seg0 - start8
        pltpu.sync_copy(
            bounds_hbm.at[pl.ds(pl.multiple_of(start8, 8), BCNT)], bvm)

        def bload(i):
            return bvm[pl.ds(i, 16)][0]

        r_lo = bload(off)
        r_hi = bload(off + SPW)
        nrows = r_hi - r_lo
        nchunks = jnp.maximum((nrows + _C - 1) // _C, 1)
        nch2 = (nchunks + 1) // 2

        pos_inf = jnp.full((16,), jnp.inf, jnp.float32)
        neg_inf = jnp.full((16,), -jnp.inf, jnp.float32)
        nbits = max(1, (SPW + 1).bit_length())
        obase = pl.multiple_of(w * OROWS, 8)

        def chunk_base(k):
            base = r_lo + k * _C
            base_c = jnp.minimum((base // 8) * 8, E - _CB)
            return base, pl.multiple_of(base_c, 8)

        def start_copy(k, buf, sem):
            _, base_c = chunk_base(k)
            pltpu.async_copy(edges_hbm.at[pl.ds(base_c, _CB)], buf, sem)

        def wait_copy(k, buf, sem):
            _, base_c = chunk_base(k)
            pltpu.make_async_copy(
                edges_hbm.at[pl.ds(base_c, _CB)], buf, sem).wait()

        def process(k, buf, st):
            s_cur, mn, mx = st
            base, base_c = chunk_base(k)
            shift = base - base_c
            n = jnp.clip(r_hi - base, 0, _C)
            limit = base + n

            def bs_body(_, lohi):
                lo, hi = lohi
                mid = (lo + hi) // 2
                c = bload(off + mid) <= limit
                return jnp.where(c, mid, lo), jnp.where(c, hi, mid)

            t_max, _ = lax.fori_loop(
                0, nbits, bs_body, (jnp.int32(0), jnp.int32(SPW + 1)))

            def row_body(j, acc):
                a, b = acc
                v = buf[shift + j]
                return jnp.minimum(a, v), jnp.maximum(b, v)

            def seg_store(s_rel, st2):
                i, mn, mx = st2[0], st2[1], st2[2]
                hi_local = st2[3] - base
                mn, mx = lax.fori_loop(i, hi_local, row_body, (mn, mx))
                obuf[s_rel // 4, pl.ds((s_rel % 4) * 32, 16)] = mn
                obuf[s_rel // 4, pl.ds((s_rel % 4) * 32 + 16, 16)] = mx
                return hi_local

            def seg_lo(s, st2):
                i, mn, mx = st2
                hi_local = seg_store(s, (i, mn, mx, bload(off + s + 1)))
                return hi_local, pos_inf, neg_inf

            def seg_hi(s, st2):
                i, mn, mx = st2
                hi_local = seg_store(s - H, (i, mn, mx, bload(off + s + 1)))
                return hi_local, pos_inf, neg_inf

            i, mn, mx = lax.fori_loop(
                s_cur, jnp.minimum(t_max, H), seg_lo,
                (jnp.int32(0), mn, mx))

            @pl.when((s_cur < H) & (t_max >= H))
            def _():
                pltpu.sync_copy(obuf, out_hbm.at[pl.ds(obase, HR)])

            i, mn, mx = lax.fori_loop(
                jnp.maximum(s_cur, H), t_max, seg_hi, (i, mn, mx))
            mn, mx = lax.fori_loop(i, n, row_body, (mn, mx))
            return t_max, mn, mx

        start_copy(0, buf0, sem0)

        def loop_body(k2, st):
            k = 2 * k2
            start_copy(k + 1, buf1, sem1)
            wait_copy(k, buf0, sem0)
            st = process(k, buf0, st)
            start_copy(k + 2, buf0, sem0)
            wait_copy(k + 1, buf1, sem1)
            st = process(k + 1, buf1, st)
            return st

        init = (jnp.int32(0), pos_inf, neg_inf)
        lax.fori_loop(0, nch2, loop_body, init)
        wait_copy(0, buf0, sem0)
        pltpu.sync_copy(obuf, out_hbm.at[pl.ds(obase + HR, HR)])

    return pool


def kernel(edge_features, cell_0_bounds):
    E, D = edge_features.shape
    S = cell_0_bounds.shape[0] - 1
    assert D == 16 and E % 8 == 0
    SPW = ((-(-S // _NW) + 63) // 64) * 64
    S_pad = _NW * SPW
    BCNT = ((SPW + 8) // 8 + 1) * 8 + 16
    pad_len = (S_pad + 48) - (S + 1)
    bounds = jnp.concatenate(
        [cell_0_bounds.astype(jnp.int32),
         jnp.full((pad_len,), E, jnp.int32)])
    out = _make_sc_pool(E, SPW, BCNT)(edge_features, bounds)
    return out.reshape(S_pad, 2 * D)[:S]
